# baseline (speedup 1.0000x reference)
import functools

import jax
import jax.numpy as jnp
from jax import lax
from jax.experimental import pallas as pl
from jax.experimental.pallas import tpu as pltpu
from jax.experimental.pallas import tpu_sc as plsc

NC = 2
NS = 16
M = 8


def _sc_mesh():
    return plsc.VectorSubcoreMesh(
        core_axis_name="c", subcore_axis_name="s", num_cores=NC, num_subcores=NS
    )


def _sc_scatter(g2, srcb, dstt, zeros16):
    NP = zeros16.shape[0]
    CH = dstt.shape[1]
    NJ = CH // M
    RPT = NP // NS

    @functools.partial(
        pl.kernel,
        out_type=jax.ShapeDtypeStruct((NC, NP, 16), jnp.float32),
        mesh=_sc_mesh(),
        compiler_params=pltpu.CompilerParams(use_tc_tiling_on_sc=False),
        scratch_types=[
            pltpu.VMEM_SHARED((NP, 16), jnp.float32),
            pltpu.VMEM((2, M, 128), jnp.int32),
            pltpu.VMEM((2, M, 128), jnp.int32),
            pltpu.VMEM((M, 128, 16), jnp.float32),
            pltpu.SemaphoreType.DMA,
            pltpu.SemaphoreType.DMA,
            pltpu.SemaphoreType.DMA((2,)),
        ],
    )
    def k(g2_h, srcb_h, dstt_h, zeros_h, out_h, acc, sv, dv, rows, semg, sems, semi):
        c = lax.axis_index("c")
        s = lax.axis_index("s")
        r0 = s * RPT
        pltpu.sync_copy(dstt_h.at[NS - 1, pl.ds(CH - M, M)], dv.at[1])
        for r in range(M):
            pltpu.async_copy(rows.at[r], acc.at[dv.at[1, r]], sems, add=True)
        pltpu.async_copy(srcb_h.at[c, s, pl.ds(0, M)], sv.at[0], semi.at[0])
        pltpu.async_copy(dstt_h.at[s, pl.ds(0, M)], dv.at[0], semi.at[0])
        pltpu.sync_copy(zeros_h.at[pl.ds(r0, RPT)], acc.at[pl.ds(r0, RPT)])
        plsc.subcore_barrier()

        def body(j, carry):
            p = lax.rem(j, 2)
            q = 1 - p
            for r in range(M):
                pltpu.make_async_copy(rows.at[r], acc.at[dv.at[q, r]], sems).wait()
            jn = lax.rem(j + 1, NJ)
            pltpu.async_copy(srcb_h.at[c, s, pl.ds(jn * M, M)], sv.at[q], semi.at[q])
            pltpu.async_copy(dstt_h.at[s, pl.ds(jn * M, M)], dv.at[q], semi.at[q])
            pltpu.make_async_copy(
                srcb_h.at[c, s, pl.ds(j * M, M)], sv.at[p], semi.at[p]
            ).wait()
            pltpu.make_async_copy(
                dstt_h.at[s, pl.ds(j * M, M)], dv.at[p], semi.at[p]
            ).wait()
            gd = [
                pltpu.async_copy(g2_h.at[sv.at[p, r]], rows.at[r], semg)
                for r in range(M)
            ]
            for r in range(M):
                gd[r].wait()
                pltpu.async_copy(rows.at[r], acc.at[dv.at[p, r]], sems, add=True)
            return carry

        lax.fori_loop(0, NJ, body, 0)
        pf = NJ % 2
        pltpu.make_async_copy(
            srcb_h.at[c, s, pl.ds(0, M)], sv.at[pf], semi.at[pf]
        ).wait()
        pltpu.make_async_copy(dstt_h.at[s, pl.ds(0, M)], dv.at[pf], semi.at[pf]).wait()
        for r in range(M):
            pltpu.make_async_copy(
                rows.at[r], acc.at[dv.at[(NJ - 1) % 2, r]], sems
            ).wait()
        plsc.subcore_barrier()
        pltpu.sync_copy(acc.at[pl.ds(r0, RPT)], out_h.at[c, pl.ds(r0, RPT)])

    return k(g2, srcb, dstt, zeros16)


def _sc_deg(dstt, zeros16, ones16):
    NP = zeros16.shape[0]
    CH = dstt.shape[1]
    HALF = CH // 2
    NJ = HALF // M
    RPT = NP // NS

    @functools.partial(
        pl.kernel,
        out_type=jax.ShapeDtypeStruct((NC, NP, 16), jnp.float32),
        mesh=_sc_mesh(),
        compiler_params=pltpu.CompilerParams(use_tc_tiling_on_sc=False),
        scratch_types=[
            pltpu.VMEM_SHARED((NP, 16), jnp.float32),
            pltpu.VMEM((2, M, 128), jnp.int32),
            pltpu.VMEM((128, 16), jnp.float32),
            pltpu.SemaphoreType.DMA,
            pltpu.SemaphoreType.DMA((2,)),
        ],
    )
    def k(dstt_h, zeros_h, ones_h, out_h, acc, dv, ones_v, sems, semi):
        c = lax.axis_index("c")
        s = lax.axis_index("s")
        r0 = s * RPT
        pltpu.sync_copy(ones_h, ones_v)
        pltpu.sync_copy(dstt_h.at[NS - 1, pl.ds(CH - M, M)], dv.at[1])
        for r in range(M):
            pltpu.async_copy(ones_v, acc.at[dv.at[1, r]], sems, add=True)
        pltpu.async_copy(dstt_h.at[s, pl.ds(c * HALF, M)], dv.at[0], semi.at[0])
        pltpu.sync_copy(zeros_h.at[pl.ds(r0, RPT)], acc.at[pl.ds(r0, RPT)])
        plsc.subcore_barrier()

        def body(j, carry):
            p = lax.rem(j, 2)
            q = 1 - p
            for r in range(M):
                pltpu.make_async_copy(ones_v, acc.at[dv.at[q, r]], sems).wait()
            jn = lax.rem(j + 1, NJ)
            pltpu.async_copy(
                dstt_h.at[s, pl.ds(c * HALF + jn * M, M)], dv.at[q], semi.at[q]
            )
            pltpu.make_async_copy(
                dstt_h.at[s, pl.ds(c * HALF + j * M, M)], dv.at[p], semi.at[p]
            ).wait()
            for r in range(M):
                pltpu.async_copy(ones_v, acc.at[dv.at[p, r]], sems, add=True)
            return carry

        lax.fori_loop(0, NJ, body, 0)
        pf = NJ % 2
        pltpu.make_async_copy(
            dstt_h.at[s, pl.ds(c * HALF, M)], dv.at[pf], semi.at[pf]
        ).wait()
        for r in range(M):
            pltpu.make_async_copy(
                ones_v, acc.at[dv.at[(NJ - 1) % 2, r]], sems
            ).wait()
        plsc.subcore_barrier()
        pltpu.sync_copy(acc.at[pl.ds(r0, RPT)], out_h.at[c, pl.ds(r0, RPT)])

    return k(dstt, zeros16, ones16)


_B8 = 512


def _tc0(deg_pack, xt_pack, w1b, b1r, wcb):
    rn = xt_pack.shape[0]

    def body(deg_r, xt_r, w1_r, b1_r, wc_r, dinv_r, g_r):
        deg = deg_r[0] + deg_r[1] + 1.0
        dinv = lax.rsqrt(deg)
        x0 = jnp.maximum(
            jnp.dot(xt_r[...], w1_r[0], preferred_element_type=jnp.float32)
            + b1_r[0, 0:1, :],
            0.0,
        )
        x1 = jnp.maximum(
            jnp.dot(xt_r[...], w1_r[1], preferred_element_type=jnp.float32)
            + b1_r[1, 0:1, :],
            0.0,
        )
        hw0 = jnp.dot(x0, wc_r[0, 0], preferred_element_type=jnp.float32) + jnp.dot(
            x1, wc_r[1, 0], preferred_element_type=jnp.float32
        )
        hw1 = jnp.dot(x0, wc_r[0, 1], preferred_element_type=jnp.float32) + jnp.dot(
            x1, wc_r[1, 1], preferred_element_type=jnp.float32
        )
        dinv_r[...] = dinv
        g_r[0] = hw0 * dinv
        g_r[1] = hw1 * dinv

    return pl.pallas_call(
        body,
        grid=(pl.cdiv(rn, _B8),),
        in_specs=[
            pl.BlockSpec((2, _B8, 128), lambda i: (0, i, 0)),
            pl.BlockSpec((_B8, 16), lambda i: (i, 0)),
            pl.BlockSpec((2, 16, 128), lambda i: (0, 0, 0)),
            pl.BlockSpec((2, 8, 128), lambda i: (0, 0, 0)),
            pl.BlockSpec((2, 2, 128, 128), lambda i: (0, 0, 0, 0)),
        ],
        out_specs=[
            pl.BlockSpec((_B8, 128), lambda i: (i, 0)),
            pl.BlockSpec((2, _B8, 128), lambda i: (0, i, 0)),
        ],
        out_shape=[
            jax.ShapeDtypeStruct((rn, 128), jnp.float32),
            jax.ShapeDtypeStruct((2, rn, 128), jnp.float32),
        ],
    )(deg_pack, xt_pack, w1b, b1r, wcb)


def _tc_mid(acc_pack, g, dinv_pack, bprev_r, wcb_next):
    rn = g.shape[1]

    def body(acc_r, g_r, dinv_r, b_r, w_r, gout_r):
        dv = dinv_r[...]
        x0 = jnp.maximum(dv * (acc_r[0] + g_r[0]) + b_r[0, 0:1, :], 0.0)
        x1 = jnp.maximum(dv * (acc_r[1] + g_r[1]) + b_r[1, 0:1, :], 0.0)
        hw0 = jnp.dot(x0, w_r[0, 0], preferred_element_type=jnp.float32) + jnp.dot(
            x1, w_r[1, 0], preferred_element_type=jnp.float32
        )
        hw1 = jnp.dot(x0, w_r[0, 1], preferred_element_type=jnp.float32) + jnp.dot(
            x1, w_r[1, 1], preferred_element_type=jnp.float32
        )
        gout_r[0] = hw0 * dv
        gout_r[1] = hw1 * dv

    return pl.pallas_call(
        body,
        grid=(pl.cdiv(rn, _B8),),
        in_specs=[
            pl.BlockSpec((2, _B8, 128), lambda i: (0, i, 0)),
            pl.BlockSpec((2, _B8, 128), lambda i: (0, i, 0)),
            pl.BlockSpec((_B8, 128), lambda i: (i, 0)),
            pl.BlockSpec((2, 8, 128), lambda i: (0, 0, 0)),
            pl.BlockSpec((2, 2, 128, 128), lambda i: (0, 0, 0, 0)),
        ],
        out_specs=pl.BlockSpec((2, _B8, 128), lambda i: (0, i, 0)),
        out_shape=jax.ShapeDtypeStruct((2, rn, 128), jnp.float32),
    )(acc_pack, g, dinv_pack, bprev_r, wcb_next)


def _tc_fin(acc_pack, g, dinv_pack, bc3r, w3b, b3r):
    rn = g.shape[1]

    def body(acc_r, g_r, dinv_r, b_r, w_r, b3_r, y_r):
        dv = dinv_r[...]
        x0 = jnp.maximum(dv * (acc_r[0] + g_r[0]) + b_r[0, 0:1, :], 0.0)
        x1 = jnp.maximum(dv * (acc_r[1] + g_r[1]) + b_r[1, 0:1, :], 0.0)
        y = jnp.dot(x0, w_r[0], preferred_element_type=jnp.float32) + jnp.dot(
            x1, w_r[1], preferred_element_type=jnp.float32
        )
        y_r[...] = y + b3_r[0:1, :]

    return pl.pallas_call(
        body,
        grid=(pl.cdiv(rn, _B8),),
        in_specs=[
            pl.BlockSpec((2, _B8, 128), lambda i: (0, i, 0)),
            pl.BlockSpec((2, _B8, 128), lambda i: (0, i, 0)),
            pl.BlockSpec((_B8, 128), lambda i: (i, 0)),
            pl.BlockSpec((2, 8, 128), lambda i: (0, 0, 0)),
            pl.BlockSpec((2, 128, 8), lambda i: (0, 0, 0)),
            pl.BlockSpec((8, 8), lambda i: (0, 0)),
        ],
        out_specs=pl.BlockSpec((_B8, 8), lambda i: (i, 0)),
        out_shape=jax.ShapeDtypeStruct((rn, 8), jnp.float32),
    )(acc_pack, g, dinv_pack, bc3r, w3b, b3r)


def _kron8(w):
    return jnp.kron(jnp.eye(8, dtype=jnp.float32), w)


def _wc_blocks(wc):
    return jnp.stack(
        [
            jnp.stack([_kron8(wc[:16, :16]), _kron8(wc[:16, 16:])]),
            jnp.stack([_kron8(wc[16:, :16]), _kron8(wc[16:, 16:])]),
        ]
    )


def _b_rows(b):
    return jnp.stack(
        [
            jnp.tile(jnp.tile(b[:16], 8)[None, :], (8, 1)),
            jnp.tile(jnp.tile(b[16:], 8)[None, :], (8, 1)),
        ]
    )


def kernel(x, t, edge_index, W1, b1, Wc1, bc1, Wc2, bc2, Wc3, bc3, W3, b3):
    n = x.shape[0]
    e = edge_index.shape[1]

    src = edge_index[0].astype(jnp.int32)
    dst = edge_index[1].astype(jnp.int32)

    ch = -(-e // (NS * 128))
    ch = -(-ch // (2 * M)) * (2 * M)
    if NS * ch * 128 - e < M * 128:
        ch += 2 * M
    ep = NS * ch * 128
    pad = ep - e
    srcp = jnp.concatenate([src, jnp.zeros((pad,), jnp.int32)])
    dstp = jnp.concatenate([dst, jnp.full((pad,), n, jnp.int32)])
    srct = srcp.reshape(NS, ch, 128)
    dstt = dstp.reshape(NS, ch, 128)
    srcb = jnp.stack([srct, srct + n])

    np_ = ((n + 128) // 128) * 128
    zeros16 = jnp.zeros((np_, 16), jnp.float32)
    ones16 = jnp.ones((128, 16), jnp.float32)

    rn = n // 8
    xt_pack = jnp.stack([x, t], axis=1).reshape(rn, 16)
    w1b = jnp.stack([_kron8(W1[:, :16]), _kron8(W1[:, 16:])])
    b1r = _b_rows(b1)
    wcb1 = _wc_blocks(Wc1)
    wcb2 = _wc_blocks(Wc2)
    wcb3 = _wc_blocks(Wc3)
    bc1r = _b_rows(bc1)
    bc2r = _b_rows(bc2)
    bc3r = _b_rows(bc3)
    w3b = jnp.stack([_kron8(W3[:16, :]), _kron8(W3[16:, :])])
    b3r = jnp.tile(b3[None, :], (8, 8))

    deg_acc = _sc_deg(dstt, zeros16, ones16)
    dinv_pack, g = _tc0(deg_acc.reshape(NC, np_ // 8, 128), xt_pack, w1b, b1r, wcb1)
    acc1 = _sc_scatter(g.reshape(2 * n, 16), srcb, dstt, zeros16)
    g = _tc_mid(acc1.reshape(NC, np_ // 8, 128), g, dinv_pack, bc1r, wcb2)
    acc2 = _sc_scatter(g.reshape(2 * n, 16), srcb, dstt, zeros16)
    g = _tc_mid(acc2.reshape(NC, np_ // 8, 128), g, dinv_pack, bc2r, wcb3)
    acc3 = _sc_scatter(g.reshape(2 * n, 16), srcb, dstt, zeros16)
    y8 = _tc_fin(acc3.reshape(NC, np_ // 8, 128), g, dinv_pack, bc3r, w3b, b3r)
    return y8.reshape(-1)


# speedup vs baseline: 1.1431x; 1.1431x over previous
import functools

import jax
import jax.numpy as jnp
from jax import lax
from jax.experimental import pallas as pl
from jax.experimental.pallas import tpu as pltpu
from jax.experimental.pallas import tpu_sc as plsc

NC = 2
NS = 16
M = 4


def _sc_mesh():
    return plsc.VectorSubcoreMesh(
        core_axis_name="c", subcore_axis_name="s", num_cores=NC, num_subcores=NS
    )


def _sc_scatter(g2, srcb, dstt, zeros16):
    NP = zeros16.shape[0]
    CH = dstt.shape[1]
    NJ = CH // M
    RPT = NP // NS

    @functools.partial(
        pl.kernel,
        out_type=jax.ShapeDtypeStruct((NC, NP, 16), jnp.float32),
        mesh=_sc_mesh(),
        compiler_params=pltpu.CompilerParams(use_tc_tiling_on_sc=False),
        scratch_types=[
            pltpu.VMEM_SHARED((NP, 16), jnp.float32),
            pltpu.VMEM((3, M, 128), jnp.int32),
            pltpu.VMEM((3, M, 128), jnp.int32),
            pltpu.VMEM((3, M, 128, 16), jnp.float32),
            pltpu.SemaphoreType.DMA((3,)),
            pltpu.SemaphoreType.DMA,
            pltpu.SemaphoreType.DMA((3,)),
        ],
    )
    def k(g2_h, srcb_h, dstt_h, zeros_h, out_h, acc, sv, dv, rows, semg, sems, semi):
        c = lax.axis_index("c")
        s = lax.axis_index("s")
        r0 = s * RPT

        def fire_idx(j, slot):
            pltpu.async_copy(
                srcb_h.at[c, s, pl.ds(j * M, M)], sv.at[slot], semi.at[slot]
            )
            pltpu.async_copy(
                dstt_h.at[s, pl.ds(j * M, M)], dv.at[slot], semi.at[slot]
            )

        def wait_idx(slot):
            pltpu.make_async_copy(
                srcb_h.at[c, s, pl.ds(0, M)], sv.at[slot], semi.at[slot]
            ).wait()
            pltpu.make_async_copy(
                dstt_h.at[s, pl.ds(0, M)], dv.at[slot], semi.at[slot]
            ).wait()

        def fire_gathers(slot):
            for r in range(M):
                pltpu.async_copy(
                    g2_h.at[sv.at[slot, r]], rows.at[slot, r], semg.at[slot]
                )

        def drain_gathers(slot):
            for r in range(M):
                pltpu.make_async_copy(
                    g2_h.at[sv.at[slot, r]], rows.at[slot, r], semg.at[slot]
                ).wait()

        def fire_scatters(slot):
            for r in range(M):
                pltpu.async_copy(
                    rows.at[slot, r], acc.at[dv.at[slot, r]], sems, add=True
                )

        def drain_scatters(slot):
            for r in range(M):
                pltpu.make_async_copy(
                    rows.at[slot, r], acc.at[dv.at[slot, r]], sems
                ).wait()

        pltpu.sync_copy(dstt_h.at[NS - 1, pl.ds(CH - M, M)], dv.at[2])
        fire_scatters(2)
        fire_idx(0, 0)
        fire_idx(1, 1)
        pltpu.sync_copy(zeros_h.at[pl.ds(r0, RPT)], acc.at[pl.ds(r0, RPT)])
        plsc.subcore_barrier()
        wait_idx(0)
        fire_gathers(0)

        def body(j, carry):
            a = lax.rem(j, 3)
            b = lax.rem(j + 1, 3)
            f = lax.rem(j + 2, 3)
            drain_scatters(f)
            fire_idx(lax.rem(j + 2, NJ), f)
            wait_idx(b)
            fire_gathers(b)
            drain_gathers(a)
            fire_scatters(a)
            return carry

        lax.fori_loop(0, NJ, body, 0)
        drain_scatters((NJ - 1) % 3)
        drain_gathers(NJ % 3)
        wait_idx((NJ + 1) % 3)
        plsc.subcore_barrier()
        pltpu.sync_copy(acc.at[pl.ds(r0, RPT)], out_h.at[c, pl.ds(r0, RPT)])

    return k(g2, srcb, dstt, zeros16)


def _sc_deg(dstt, zeros16, ones16):
    NP = zeros16.shape[0]
    CH = dstt.shape[1]
    HALF = CH // 2
    NJ = HALF // M
    RPT = NP // NS

    @functools.partial(
        pl.kernel,
        out_type=jax.ShapeDtypeStruct((NC, NP, 16), jnp.float32),
        mesh=_sc_mesh(),
        compiler_params=pltpu.CompilerParams(use_tc_tiling_on_sc=False),
        scratch_types=[
            pltpu.VMEM_SHARED((NP, 16), jnp.float32),
            pltpu.VMEM((2, M, 128), jnp.int32),
            pltpu.VMEM((128, 16), jnp.float32),
            pltpu.SemaphoreType.DMA,
            pltpu.SemaphoreType.DMA((2,)),
        ],
    )
    def k(dstt_h, zeros_h, ones_h, out_h, acc, dv, ones_v, sems, semi):
        c = lax.axis_index("c")
        s = lax.axis_index("s")
        r0 = s * RPT
        pltpu.sync_copy(ones_h, ones_v)
        pltpu.sync_copy(dstt_h.at[NS - 1, pl.ds(CH - M, M)], dv.at[1])
        for r in range(M):
            pltpu.async_copy(ones_v, acc.at[dv.at[1, r]], sems, add=True)
        pltpu.async_copy(dstt_h.at[s, pl.ds(c * HALF, M)], dv.at[0], semi.at[0])
        pltpu.sync_copy(zeros_h.at[pl.ds(r0, RPT)], acc.at[pl.ds(r0, RPT)])
        plsc.subcore_barrier()

        def body(j, carry):
            p = lax.rem(j, 2)
            q = 1 - p
            for r in range(M):
                pltpu.make_async_copy(ones_v, acc.at[dv.at[q, r]], sems).wait()
            jn = lax.rem(j + 1, NJ)
            pltpu.async_copy(
                dstt_h.at[s, pl.ds(c * HALF + jn * M, M)], dv.at[q], semi.at[q]
            )
            pltpu.make_async_copy(
                dstt_h.at[s, pl.ds(c * HALF + j * M, M)], dv.at[p], semi.at[p]
            ).wait()
            for r in range(M):
                pltpu.async_copy(ones_v, acc.at[dv.at[p, r]], sems, add=True)
            return carry

        lax.fori_loop(0, NJ, body, 0)
        pf = NJ % 2
        pltpu.make_async_copy(
            dstt_h.at[s, pl.ds(c * HALF, M)], dv.at[pf], semi.at[pf]
        ).wait()
        for r in range(M):
            pltpu.make_async_copy(
                ones_v, acc.at[dv.at[(NJ - 1) % 2, r]], sems
            ).wait()
        plsc.subcore_barrier()
        pltpu.sync_copy(acc.at[pl.ds(r0, RPT)], out_h.at[c, pl.ds(r0, RPT)])

    return k(dstt, zeros16, ones16)


_B8 = 512


def _tc0(deg_pack, xt_pack, w1b, b1r, wcb):
    rn = xt_pack.shape[0]

    def body(deg_r, xt_r, w1_r, b1_r, wc_r, dinv_r, g_r):
        deg = deg_r[0] + deg_r[1] + 1.0
        dinv = lax.rsqrt(deg)
        x0 = jnp.maximum(
            jnp.dot(xt_r[...], w1_r[0], preferred_element_type=jnp.float32)
            + b1_r[0, 0:1, :],
            0.0,
        )
        x1 = jnp.maximum(
            jnp.dot(xt_r[...], w1_r[1], preferred_element_type=jnp.float32)
            + b1_r[1, 0:1, :],
            0.0,
        )
        hw0 = jnp.dot(x0, wc_r[0, 0], preferred_element_type=jnp.float32) + jnp.dot(
            x1, wc_r[1, 0], preferred_element_type=jnp.float32
        )
        hw1 = jnp.dot(x0, wc_r[0, 1], preferred_element_type=jnp.float32) + jnp.dot(
            x1, wc_r[1, 1], preferred_element_type=jnp.float32
        )
        dinv_r[...] = dinv
        g_r[0] = hw0 * dinv
        g_r[1] = hw1 * dinv

    return pl.pallas_call(
        body,
        grid=(pl.cdiv(rn, _B8),),
        in_specs=[
            pl.BlockSpec((2, _B8, 128), lambda i: (0, i, 0)),
            pl.BlockSpec((_B8, 16), lambda i: (i, 0)),
            pl.BlockSpec((2, 16, 128), lambda i: (0, 0, 0)),
            pl.BlockSpec((2, 8, 128), lambda i: (0, 0, 0)),
            pl.BlockSpec((2, 2, 128, 128), lambda i: (0, 0, 0, 0)),
        ],
        out_specs=[
            pl.BlockSpec((_B8, 128), lambda i: (i, 0)),
            pl.BlockSpec((2, _B8, 128), lambda i: (0, i, 0)),
        ],
        out_shape=[
            jax.ShapeDtypeStruct((rn, 128), jnp.float32),
            jax.ShapeDtypeStruct((2, rn, 128), jnp.float32),
        ],
    )(deg_pack, xt_pack, w1b, b1r, wcb)


def _tc_mid(acc_pack, g, dinv_pack, bprev_r, wcb_next):
    rn = g.shape[1]

    def body(acc_r, g_r, dinv_r, b_r, w_r, gout_r):
        dv = dinv_r[...]
        x0 = jnp.maximum(dv * (acc_r[0] + g_r[0]) + b_r[0, 0:1, :], 0.0)
        x1 = jnp.maximum(dv * (acc_r[1] + g_r[1]) + b_r[1, 0:1, :], 0.0)
        hw0 = jnp.dot(x0, w_r[0, 0], preferred_element_type=jnp.float32) + jnp.dot(
            x1, w_r[1, 0], preferred_element_type=jnp.float32
        )
        hw1 = jnp.dot(x0, w_r[0, 1], preferred_element_type=jnp.float32) + jnp.dot(
            x1, w_r[1, 1], preferred_element_type=jnp.float32
        )
        gout_r[0] = hw0 * dv
        gout_r[1] = hw1 * dv

    return pl.pallas_call(
        body,
        grid=(pl.cdiv(rn, _B8),),
        in_specs=[
            pl.BlockSpec((2, _B8, 128), lambda i: (0, i, 0)),
            pl.BlockSpec((2, _B8, 128), lambda i: (0, i, 0)),
            pl.BlockSpec((_B8, 128), lambda i: (i, 0)),
            pl.BlockSpec((2, 8, 128), lambda i: (0, 0, 0)),
            pl.BlockSpec((2, 2, 128, 128), lambda i: (0, 0, 0, 0)),
        ],
        out_specs=pl.BlockSpec((2, _B8, 128), lambda i: (0, i, 0)),
        out_shape=jax.ShapeDtypeStruct((2, rn, 128), jnp.float32),
    )(acc_pack, g, dinv_pack, bprev_r, wcb_next)


def _tc_fin(acc_pack, g, dinv_pack, bc3r, w3b, b3r):
    rn = g.shape[1]

    def body(acc_r, g_r, dinv_r, b_r, w_r, b3_r, y_r):
        dv = dinv_r[...]
        x0 = jnp.maximum(dv * (acc_r[0] + g_r[0]) + b_r[0, 0:1, :], 0.0)
        x1 = jnp.maximum(dv * (acc_r[1] + g_r[1]) + b_r[1, 0:1, :], 0.0)
        y = jnp.dot(x0, w_r[0], preferred_element_type=jnp.float32) + jnp.dot(
            x1, w_r[1], preferred_element_type=jnp.float32
        )
        y_r[...] = y + b3_r[0:1, :]

    return pl.pallas_call(
        body,
        grid=(pl.cdiv(rn, _B8),),
        in_specs=[
            pl.BlockSpec((2, _B8, 128), lambda i: (0, i, 0)),
            pl.BlockSpec((2, _B8, 128), lambda i: (0, i, 0)),
            pl.BlockSpec((_B8, 128), lambda i: (i, 0)),
            pl.BlockSpec((2, 8, 128), lambda i: (0, 0, 0)),
            pl.BlockSpec((2, 128, 8), lambda i: (0, 0, 0)),
            pl.BlockSpec((8, 8), lambda i: (0, 0)),
        ],
        out_specs=pl.BlockSpec((_B8, 8), lambda i: (i, 0)),
        out_shape=jax.ShapeDtypeStruct((rn, 8), jnp.float32),
    )(acc_pack, g, dinv_pack, bc3r, w3b, b3r)


def _kron8(w):
    return jnp.kron(jnp.eye(8, dtype=jnp.float32), w)


def _wc_blocks(wc):
    return jnp.stack(
        [
            jnp.stack([_kron8(wc[:16, :16]), _kron8(wc[:16, 16:])]),
            jnp.stack([_kron8(wc[16:, :16]), _kron8(wc[16:, 16:])]),
        ]
    )


def _b_rows(b):
    return jnp.stack(
        [
            jnp.tile(jnp.tile(b[:16], 8)[None, :], (8, 1)),
            jnp.tile(jnp.tile(b[16:], 8)[None, :], (8, 1)),
        ]
    )


def kernel(x, t, edge_index, W1, b1, Wc1, bc1, Wc2, bc2, Wc3, bc3, W3, b3):
    n = x.shape[0]
    e = edge_index.shape[1]

    src = edge_index[0].astype(jnp.int32)
    dst = edge_index[1].astype(jnp.int32)

    ch = -(-e // (NS * 128))
    ch = -(-ch // (2 * M)) * (2 * M)
    if NS * ch * 128 - e < M * 128:
        ch += 2 * M
    ep = NS * ch * 128
    pad = ep - e
    srcp = jnp.concatenate([src, jnp.zeros((pad,), jnp.int32)])
    dstp = jnp.concatenate([dst, jnp.full((pad,), n, jnp.int32)])
    srct = srcp.reshape(NS, ch, 128)
    dstt = dstp.reshape(NS, ch, 128)
    srcb = jnp.stack([srct, srct + n])

    np_ = ((n + 128) // 128) * 128
    zeros16 = jnp.zeros((np_, 16), jnp.float32)
    ones16 = jnp.ones((128, 16), jnp.float32)

    rn = n // 8
    xt_pack = jnp.stack([x, t], axis=1).reshape(rn, 16)
    w1b = jnp.stack([_kron8(W1[:, :16]), _kron8(W1[:, 16:])])
    b1r = _b_rows(b1)
    wcb1 = _wc_blocks(Wc1)
    wcb2 = _wc_blocks(Wc2)
    wcb3 = _wc_blocks(Wc3)
    bc1r = _b_rows(bc1)
    bc2r = _b_rows(bc2)
    bc3r = _b_rows(bc3)
    w3b = jnp.stack([_kron8(W3[:16, :]), _kron8(W3[16:, :])])
    b3r = jnp.tile(b3[None, :], (8, 8))

    deg_acc = _sc_deg(dstt, zeros16, ones16)
    dinv_pack, g = _tc0(deg_acc.reshape(NC, np_ // 8, 128), xt_pack, w1b, b1r, wcb1)
    acc1 = _sc_scatter(g.reshape(2 * n, 16), srcb, dstt, zeros16)
    g = _tc_mid(acc1.reshape(NC, np_ // 8, 128), g, dinv_pack, bc1r, wcb2)
    acc2 = _sc_scatter(g.reshape(2 * n, 16), srcb, dstt, zeros16)
    g = _tc_mid(acc2.reshape(NC, np_ // 8, 128), g, dinv_pack, bc2r, wcb3)
    acc3 = _sc_scatter(g.reshape(2 * n, 16), srcb, dstt, zeros16)
    y8 = _tc_fin(acc3.reshape(NC, np_ // 8, 128), g, dinv_pack, bc3r, w3b, b3r)
    return y8.reshape(-1)


# speedup vs baseline: 1.2029x; 1.0523x over previous
import functools

import jax
import jax.numpy as jnp
from jax import lax
from jax.experimental import pallas as pl
from jax.experimental.pallas import tpu as pltpu
from jax.experimental.pallas import tpu_sc as plsc

NC = 2
NS = 16
M = 4


def _sc_mesh():
    return plsc.VectorSubcoreMesh(
        core_axis_name="c", subcore_axis_name="s", num_cores=NC, num_subcores=NS
    )


def _sc_scatter(g2, srcb, dstt, zeros16):
    NP = zeros16.shape[0]
    CH = dstt.shape[1]
    NJ = CH // M
    RPT = NP // NS

    @functools.partial(
        pl.kernel,
        out_type=jax.ShapeDtypeStruct((NC, NP, 16), jnp.float32),
        mesh=_sc_mesh(),
        compiler_params=pltpu.CompilerParams(use_tc_tiling_on_sc=False),
        scratch_types=[
            pltpu.VMEM_SHARED((NP, 16), jnp.float32),
            pltpu.VMEM((3, M, 128), jnp.int32),
            pltpu.VMEM((3, M, 128), jnp.int32),
            pltpu.VMEM((3, M, 128, 16), jnp.float32),
            pltpu.SemaphoreType.DMA((3,)),
            pltpu.SemaphoreType.DMA,
            pltpu.SemaphoreType.DMA((3,)),
        ],
    )
    def k(g2_h, srcb_h, dstt_h, zeros_h, out_h, acc, sv, dv, rows, semg, sems, semi):
        c = lax.axis_index("c")
        s = lax.axis_index("s")
        r0 = s * RPT

        def fire_idx(j, slot):
            pltpu.async_copy(
                srcb_h.at[c, s, pl.ds(j * M, M)], sv.at[slot], semi.at[slot]
            )
            pltpu.async_copy(
                dstt_h.at[s, pl.ds(j * M, M)], dv.at[slot], semi.at[slot]
            )

        def wait_idx(slot):
            pltpu.make_async_copy(
                srcb_h.at[c, s, pl.ds(0, M)], sv.at[slot], semi.at[slot]
            ).wait()
            pltpu.make_async_copy(
                dstt_h.at[s, pl.ds(0, M)], dv.at[slot], semi.at[slot]
            ).wait()

        def fire_gathers(slot):
            for r in range(M):
                pltpu.async_copy(
                    g2_h.at[sv.at[slot, r]], rows.at[slot, r], semg.at[slot]
                )

        def drain_gathers(slot):
            for r in range(M):
                pltpu.make_async_copy(
                    g2_h.at[sv.at[slot, r]], rows.at[slot, r], semg.at[slot]
                ).wait()

        def fire_scatters(slot):
            for r in range(M):
                pltpu.async_copy(
                    rows.at[slot, r], acc.at[dv.at[slot, r]], sems, add=True
                )

        def drain_scatters(slot):
            for r in range(M):
                pltpu.make_async_copy(
                    rows.at[slot, r], acc.at[dv.at[slot, r]], sems
                ).wait()

        pltpu.sync_copy(dstt_h.at[NS - 1, pl.ds(CH - M, M)], dv.at[2])
        fire_scatters(2)
        fire_idx(0, 0)
        fire_idx(1, 1)
        pltpu.sync_copy(zeros_h.at[pl.ds(r0, RPT)], acc.at[pl.ds(r0, RPT)])
        plsc.subcore_barrier()
        wait_idx(0)
        fire_gathers(0)

        def body(j, carry):
            a = lax.rem(j, 3)
            b = lax.rem(j + 1, 3)
            f = lax.rem(j + 2, 3)
            drain_scatters(f)
            fire_idx(lax.rem(j + 2, NJ), f)
            wait_idx(b)
            fire_gathers(b)
            for r in range(M):
                pltpu.make_async_copy(
                    g2_h.at[sv.at[a, r]], rows.at[a, r], semg.at[a]
                ).wait()
                pltpu.async_copy(
                    rows.at[a, r], acc.at[dv.at[a, r]], sems, add=True
                )
            return carry

        lax.fori_loop(0, NJ, body, 0)
        drain_scatters((NJ - 1) % 3)
        drain_gathers(NJ % 3)
        wait_idx((NJ + 1) % 3)
        plsc.subcore_barrier()
        pltpu.sync_copy(acc.at[pl.ds(r0, RPT)], out_h.at[c, pl.ds(r0, RPT)])

    return k(g2, srcb, dstt, zeros16)


def _sc_deg(dstt, zeros16, ones16):
    NP = zeros16.shape[0]
    CH = dstt.shape[1]
    HALF = CH // 2
    MD = 8
    NJ = HALF // MD
    RPT = NP // NS

    @functools.partial(
        pl.kernel,
        out_type=jax.ShapeDtypeStruct((NC, NP, 16), jnp.float32),
        mesh=_sc_mesh(),
        compiler_params=pltpu.CompilerParams(use_tc_tiling_on_sc=False),
        scratch_types=[
            pltpu.VMEM_SHARED((NP, 16), jnp.float32),
            pltpu.VMEM((2, MD, 128), jnp.int32),
            pltpu.VMEM((128, 16), jnp.float32),
            pltpu.SemaphoreType.DMA,
            pltpu.SemaphoreType.DMA((2,)),
        ],
    )
    def k(dstt_h, zeros_h, ones_h, out_h, acc, dv, ones_v, sems, semi):
        c = lax.axis_index("c")
        s = lax.axis_index("s")
        r0 = s * RPT
        pltpu.sync_copy(ones_h, ones_v)
        pltpu.sync_copy(dstt_h.at[NS - 1, pl.ds(CH - MD, MD)], dv.at[1])
        for r in range(MD):
            pltpu.async_copy(ones_v, acc.at[dv.at[1, r]], sems, add=True)
        pltpu.async_copy(dstt_h.at[s, pl.ds(c * HALF, MD)], dv.at[0], semi.at[0])
        pltpu.sync_copy(zeros_h.at[pl.ds(r0, RPT)], acc.at[pl.ds(r0, RPT)])
        plsc.subcore_barrier()

        def body(j, carry):
            p = lax.rem(j, 2)
            q = 1 - p
            for r in range(MD):
                pltpu.make_async_copy(ones_v, acc.at[dv.at[q, r]], sems).wait()
            jn = lax.rem(j + 1, NJ)
            pltpu.async_copy(
                dstt_h.at[s, pl.ds(c * HALF + jn * MD, MD)], dv.at[q], semi.at[q]
            )
            pltpu.make_async_copy(
                dstt_h.at[s, pl.ds(c * HALF + j * MD, MD)], dv.at[p], semi.at[p]
            ).wait()
            for r in range(MD):
                pltpu.async_copy(ones_v, acc.at[dv.at[p, r]], sems, add=True)
            return carry

        lax.fori_loop(0, NJ, body, 0)
        pf = NJ % 2
        pltpu.make_async_copy(
            dstt_h.at[s, pl.ds(c * HALF, MD)], dv.at[pf], semi.at[pf]
        ).wait()
        for r in range(MD):
            pltpu.make_async_copy(
                ones_v, acc.at[dv.at[(NJ - 1) % 2, r]], sems
            ).wait()
        plsc.subcore_barrier()
        pltpu.sync_copy(acc.at[pl.ds(r0, RPT)], out_h.at[c, pl.ds(r0, RPT)])

    return k(dstt, zeros16, ones16)


_B8 = 512


def _tc0(deg_pack, xt_pack, w1b, b1r, wcb):
    rn = xt_pack.shape[0]

    def body(deg_r, xt_r, w1_r, b1_r, wc_r, dinv_r, g_r):
        deg = deg_r[0] + deg_r[1] + 1.0
        dinv = lax.rsqrt(deg)
        x0 = jnp.maximum(
            jnp.dot(xt_r[...], w1_r[0], preferred_element_type=jnp.float32)
            + b1_r[0, 0:1, :],
            0.0,
        )
        x1 = jnp.maximum(
            jnp.dot(xt_r[...], w1_r[1], preferred_element_type=jnp.float32)
            + b1_r[1, 0:1, :],
            0.0,
        )
        hw0 = jnp.dot(x0, wc_r[0, 0], preferred_element_type=jnp.float32) + jnp.dot(
            x1, wc_r[1, 0], preferred_element_type=jnp.float32
        )
        hw1 = jnp.dot(x0, wc_r[0, 1], preferred_element_type=jnp.float32) + jnp.dot(
            x1, wc_r[1, 1], preferred_element_type=jnp.float32
        )
        dinv_r[...] = dinv
        g_r[0] = hw0 * dinv
        g_r[1] = hw1 * dinv

    return pl.pallas_call(
        body,
        grid=(pl.cdiv(rn, _B8),),
        in_specs=[
            pl.BlockSpec((2, _B8, 128), lambda i: (0, i, 0)),
            pl.BlockSpec((_B8, 16), lambda i: (i, 0)),
            pl.BlockSpec((2, 16, 128), lambda i: (0, 0, 0)),
            pl.BlockSpec((2, 8, 128), lambda i: (0, 0, 0)),
            pl.BlockSpec((2, 2, 128, 128), lambda i: (0, 0, 0, 0)),
        ],
        out_specs=[
            pl.BlockSpec((_B8, 128), lambda i: (i, 0)),
            pl.BlockSpec((2, _B8, 128), lambda i: (0, i, 0)),
        ],
        out_shape=[
            jax.ShapeDtypeStruct((rn, 128), jnp.float32),
            jax.ShapeDtypeStruct((2, rn, 128), jnp.float32),
        ],
    )(deg_pack, xt_pack, w1b, b1r, wcb)


def _tc_mid(acc_pack, g, dinv_pack, bprev_r, wcb_next):
    rn = g.shape[1]

    def body(acc_r, g_r, dinv_r, b_r, w_r, gout_r):
        dv = dinv_r[...]
        x0 = jnp.maximum(dv * (acc_r[0] + g_r[0]) + b_r[0, 0:1, :], 0.0)
        x1 = jnp.maximum(dv * (acc_r[1] + g_r[1]) + b_r[1, 0:1, :], 0.0)
        hw0 = jnp.dot(x0, w_r[0, 0], preferred_element_type=jnp.float32) + jnp.dot(
            x1, w_r[1, 0], preferred_element_type=jnp.float32
        )
        hw1 = jnp.dot(x0, w_r[0, 1], preferred_element_type=jnp.float32) + jnp.dot(
            x1, w_r[1, 1], preferred_element_type=jnp.float32
        )
        gout_r[0] = hw0 * dv
        gout_r[1] = hw1 * dv

    return pl.pallas_call(
        body,
        grid=(pl.cdiv(rn, _B8),),
        in_specs=[
            pl.BlockSpec((2, _B8, 128), lambda i: (0, i, 0)),
            pl.BlockSpec((2, _B8, 128), lambda i: (0, i, 0)),
            pl.BlockSpec((_B8, 128), lambda i: (i, 0)),
            pl.BlockSpec((2, 8, 128), lambda i: (0, 0, 0)),
            pl.BlockSpec((2, 2, 128, 128), lambda i: (0, 0, 0, 0)),
        ],
        out_specs=pl.BlockSpec((2, _B8, 128), lambda i: (0, i, 0)),
        out_shape=jax.ShapeDtypeStruct((2, rn, 128), jnp.float32),
    )(acc_pack, g, dinv_pack, bprev_r, wcb_next)


def _tc_fin(acc_pack, g, dinv_pack, bc3r, w3b, b3r):
    rn = g.shape[1]

    def body(acc_r, g_r, dinv_r, b_r, w_r, b3_r, y_r):
        dv = dinv_r[...]
        x0 = jnp.maximum(dv * (acc_r[0] + g_r[0]) + b_r[0, 0:1, :], 0.0)
        x1 = jnp.maximum(dv * (acc_r[1] + g_r[1]) + b_r[1, 0:1, :], 0.0)
        y = jnp.dot(x0, w_r[0], preferred_element_type=jnp.float32) + jnp.dot(
            x1, w_r[1], preferred_element_type=jnp.float32
        )
        y_r[...] = y + b3_r[0:1, :]

    return pl.pallas_call(
        body,
        grid=(pl.cdiv(rn, _B8),),
        in_specs=[
            pl.BlockSpec((2, _B8, 128), lambda i: (0, i, 0)),
            pl.BlockSpec((2, _B8, 128), lambda i: (0, i, 0)),
            pl.BlockSpec((_B8, 128), lambda i: (i, 0)),
            pl.BlockSpec((2, 8, 128), lambda i: (0, 0, 0)),
            pl.BlockSpec((2, 128, 8), lambda i: (0, 0, 0)),
            pl.BlockSpec((8, 8), lambda i: (0, 0)),
        ],
        out_specs=pl.BlockSpec((_B8, 8), lambda i: (i, 0)),
        out_shape=jax.ShapeDtypeStruct((rn, 8), jnp.float32),
    )(acc_pack, g, dinv_pack, bc3r, w3b, b3r)


def _kron8(w):
    return jnp.kron(jnp.eye(8, dtype=jnp.float32), w)


def _wc_blocks(wc):
    return jnp.stack(
        [
            jnp.stack([_kron8(wc[:16, :16]), _kron8(wc[:16, 16:])]),
            jnp.stack([_kron8(wc[16:, :16]), _kron8(wc[16:, 16:])]),
        ]
    )


def _b_rows(b):
    return jnp.stack(
        [
            jnp.tile(jnp.tile(b[:16], 8)[None, :], (8, 1)),
            jnp.tile(jnp.tile(b[16:], 8)[None, :], (8, 1)),
        ]
    )


def kernel(x, t, edge_index, W1, b1, Wc1, bc1, Wc2, bc2, Wc3, bc3, W3, b3):
    n = x.shape[0]
    e = edge_index.shape[1]

    src = edge_index[0].astype(jnp.int32)
    dst = edge_index[1].astype(jnp.int32)

    ch = -(-e // (NS * 128))
    ch = -(-ch // (2 * M)) * (2 * M)
    if NS * ch * 128 - e < M * 128:
        ch += 2 * M
    ep = NS * ch * 128
    pad = ep - e
    srcp = jnp.concatenate([src, jnp.zeros((pad,), jnp.int32)])
    dstp = jnp.concatenate([dst, jnp.full((pad,), n, jnp.int32)])
    srct = srcp.reshape(NS, ch, 128)
    dstt = dstp.reshape(NS, ch, 128)
    srcb = jnp.stack([srct, srct + n])

    np_ = ((n + 128) // 128) * 128
    zeros16 = jnp.zeros((np_, 16), jnp.float32)
    ones16 = jnp.ones((128, 16), jnp.float32)

    rn = n // 8
    xt_pack = jnp.stack([x, t], axis=1).reshape(rn, 16)
    w1b = jnp.stack([_kron8(W1[:, :16]), _kron8(W1[:, 16:])])
    b1r = _b_rows(b1)
    wcb1 = _wc_blocks(Wc1)
    wcb2 = _wc_blocks(Wc2)
    wcb3 = _wc_blocks(Wc3)
    bc1r = _b_rows(bc1)
    bc2r = _b_rows(bc2)
    bc3r = _b_rows(bc3)
    w3b = jnp.stack([_kron8(W3[:16, :]), _kron8(W3[16:, :])])
    b3r = jnp.tile(b3[None, :], (8, 8))

    deg_acc = _sc_deg(dstt, zeros16, ones16)
    dinv_pack, g = _tc0(deg_acc.reshape(NC, np_ // 8, 128), xt_pack, w1b, b1r, wcb1)
    acc1 = _sc_scatter(g.reshape(2 * n, 16), srcb, dstt, zeros16)
    g = _tc_mid(acc1.reshape(NC, np_ // 8, 128), g, dinv_pack, bc1r, wcb2)
    acc2 = _sc_scatter(g.reshape(2 * n, 16), srcb, dstt, zeros16)
    g = _tc_mid(acc2.reshape(NC, np_ // 8, 128), g, dinv_pack, bc2r, wcb3)
    acc3 = _sc_scatter(g.reshape(2 * n, 16), srcb, dstt, zeros16)
    y8 = _tc_fin(acc3.reshape(NC, np_ // 8, 128), g, dinv_pack, bc3r, w3b, b3r)
    return y8.reshape(-1)


# speedup vs baseline: 1.2437x; 1.0339x over previous
import functools

import jax
import jax.numpy as jnp
from jax import lax
from jax.experimental import pallas as pl
from jax.experimental.pallas import tpu as pltpu
from jax.experimental.pallas import tpu_sc as plsc

NC = 2
NS = 16
M = 4


def _sc_mesh():
    return plsc.VectorSubcoreMesh(
        core_axis_name="c", subcore_axis_name="s", num_cores=NC, num_subcores=NS
    )


def _sc_scatter(g2, srcb, dstt, zeros16):
    NP = zeros16.shape[0]
    CH = dstt.shape[1]
    NJ = CH // M
    RPT = NP // NS

    @functools.partial(
        pl.kernel,
        out_type=jax.ShapeDtypeStruct((NC, NP, 16), jnp.float32),
        mesh=_sc_mesh(),
        compiler_params=pltpu.CompilerParams(use_tc_tiling_on_sc=False),
        scratch_types=[
            pltpu.VMEM_SHARED((NP, 16), jnp.float32),
            pltpu.VMEM((3, M, 128), jnp.int32),
            pltpu.VMEM((3, M, 128), jnp.int32),
            pltpu.VMEM((3, M, 128, 16), jnp.float32),
            pltpu.SemaphoreType.DMA((3,)),
            pltpu.SemaphoreType.DMA,
            pltpu.SemaphoreType.DMA((3,)),
        ],
    )
    def k(g2_h, srcb_h, dstt_h, zeros_h, out_h, acc, sv, dv, rows, semg, sems, semi):
        c = lax.axis_index("c")
        s = lax.axis_index("s")
        r0 = s * RPT

        def fire_idx(j, slot):
            pltpu.async_copy(
                srcb_h.at[c, s, pl.ds(j * M, M)], sv.at[slot], semi.at[slot]
            )
            pltpu.async_copy(
                dstt_h.at[s, pl.ds(j * M, M)], dv.at[slot], semi.at[slot]
            )

        def wait_idx(slot):
            pltpu.make_async_copy(
                srcb_h.at[c, s, pl.ds(0, M)], sv.at[slot], semi.at[slot]
            ).wait()
            pltpu.make_async_copy(
                dstt_h.at[s, pl.ds(0, M)], dv.at[slot], semi.at[slot]
            ).wait()

        def fire_gathers(slot):
            for r in range(M):
                pltpu.async_copy(
                    g2_h.at[sv.at[slot, r]], rows.at[slot, r], semg.at[slot]
                )

        def drain_gathers(slot):
            for r in range(M):
                pltpu.make_async_copy(
                    g2_h.at[sv.at[slot, r]], rows.at[slot, r], semg.at[slot]
                ).wait()

        def fire_scatters(slot):
            for r in range(M):
                pltpu.async_copy(
                    rows.at[slot, r], acc.at[dv.at[slot, r]], sems, add=True
                )

        def drain_scatters(slot):
            for r in range(M):
                pltpu.make_async_copy(
                    rows.at[slot, r], acc.at[dv.at[slot, r]], sems
                ).wait()

        pltpu.sync_copy(dstt_h.at[NS - 1, pl.ds(CH - M, M)], dv.at[2])
        fire_scatters(2)
        fire_idx(0, 0)
        fire_idx(1, 1)
        pltpu.sync_copy(zeros_h.at[pl.ds(r0, RPT)], acc.at[pl.ds(r0, RPT)])
        plsc.subcore_barrier()
        wait_idx(0)
        fire_gathers(0)

        def body(j, carry):
            a = lax.rem(j, 3)
            b = lax.rem(j + 1, 3)
            f = lax.rem(j + 2, 3)
            drain_scatters(f)
            fire_idx(lax.rem(j + 2, NJ), f)
            wait_idx(b)
            fire_gathers(b)
            for r in range(M):
                pltpu.make_async_copy(
                    g2_h.at[sv.at[a, r]], rows.at[a, r], semg.at[a]
                ).wait()
                pltpu.async_copy(
                    rows.at[a, r], acc.at[dv.at[a, r]], sems, add=True
                )
            return carry

        lax.fori_loop(0, NJ, body, 0)
        drain_scatters((NJ - 1) % 3)
        drain_gathers(NJ % 3)
        wait_idx((NJ + 1) % 3)
        plsc.subcore_barrier()
        pltpu.sync_copy(acc.at[pl.ds(r0, RPT)], out_h.at[c, pl.ds(r0, RPT)])

    return k(g2, srcb, dstt, zeros16)


def _sc_deg(dstt, zeros16, ones16):
    NP = zeros16.shape[0]
    CH = dstt.shape[1]
    HALF = CH // 2
    MD = 8
    NJ = HALF // MD
    RPT = NP // NS

    @functools.partial(
        pl.kernel,
        out_type=jax.ShapeDtypeStruct((NC, NP, 16), jnp.float32),
        mesh=_sc_mesh(),
        compiler_params=pltpu.CompilerParams(use_tc_tiling_on_sc=False),
        scratch_types=[
            pltpu.VMEM_SHARED((NP, 16), jnp.float32),
            pltpu.VMEM((2, MD, 128), jnp.int32),
            pltpu.VMEM((128, 16), jnp.float32),
            pltpu.SemaphoreType.DMA,
            pltpu.SemaphoreType.DMA((2,)),
        ],
    )
    def k(dstt_h, zeros_h, ones_h, out_h, acc, dv, ones_v, sems, semi):
        c = lax.axis_index("c")
        s = lax.axis_index("s")
        r0 = s * RPT
        pltpu.sync_copy(ones_h, ones_v)
        pltpu.sync_copy(dstt_h.at[NS - 1, pl.ds(CH - MD, MD)], dv.at[1])
        for r in range(MD):
            pltpu.async_copy(ones_v, acc.at[dv.at[1, r]], sems, add=True)
        pltpu.async_copy(dstt_h.at[s, pl.ds(c * HALF, MD)], dv.at[0], semi.at[0])
        pltpu.sync_copy(zeros_h.at[pl.ds(r0, RPT)], acc.at[pl.ds(r0, RPT)])
        plsc.subcore_barrier()

        def body(j, carry):
            p = lax.rem(j, 2)
            q = 1 - p
            for r in range(MD):
                pltpu.make_async_copy(ones_v, acc.at[dv.at[q, r]], sems).wait()
            jn = lax.rem(j + 1, NJ)
            pltpu.async_copy(
                dstt_h.at[s, pl.ds(c * HALF + jn * MD, MD)], dv.at[q], semi.at[q]
            )
            pltpu.make_async_copy(
                dstt_h.at[s, pl.ds(c * HALF + j * MD, MD)], dv.at[p], semi.at[p]
            ).wait()
            for r in range(MD):
                pltpu.async_copy(ones_v, acc.at[dv.at[p, r]], sems, add=True)
            return carry

        lax.fori_loop(0, NJ, body, 0)
        pf = NJ % 2
        pltpu.make_async_copy(
            dstt_h.at[s, pl.ds(c * HALF, MD)], dv.at[pf], semi.at[pf]
        ).wait()
        for r in range(MD):
            pltpu.make_async_copy(
                ones_v, acc.at[dv.at[(NJ - 1) % 2, r]], sems
            ).wait()
        plsc.subcore_barrier()
        pltpu.sync_copy(acc.at[pl.ds(r0, RPT)], out_h.at[c, pl.ds(r0, RPT)])

    return k(dstt, zeros16, ones16)


_B8 = 1024


def _tc0(deg_pack, xt_pack, w1b, b1r, wcb):
    rn = xt_pack.shape[0]

    def body(deg_r, xt_r, w1_r, b1_r, wc_r, dinv_r, g_r):
        deg = deg_r[0] + deg_r[1] + 1.0
        dinv = lax.rsqrt(deg)
        x0 = jnp.maximum(
            jnp.dot(xt_r[...], w1_r[0], preferred_element_type=jnp.float32)
            + b1_r[0, 0:1, :],
            0.0,
        )
        x1 = jnp.maximum(
            jnp.dot(xt_r[...], w1_r[1], preferred_element_type=jnp.float32)
            + b1_r[1, 0:1, :],
            0.0,
        )
        hw0 = jnp.dot(x0, wc_r[0, 0], preferred_element_type=jnp.float32) + jnp.dot(
            x1, wc_r[1, 0], preferred_element_type=jnp.float32
        )
        hw1 = jnp.dot(x0, wc_r[0, 1], preferred_element_type=jnp.float32) + jnp.dot(
            x1, wc_r[1, 1], preferred_element_type=jnp.float32
        )
        dinv_r[...] = dinv
        g_r[0] = hw0 * dinv
        g_r[1] = hw1 * dinv

    return pl.pallas_call(
        body,
        grid=(pl.cdiv(rn, _B8),),
        in_specs=[
            pl.BlockSpec((2, _B8, 128), lambda i: (0, i, 0)),
            pl.BlockSpec((_B8, 16), lambda i: (i, 0)),
            pl.BlockSpec((2, 16, 128), lambda i: (0, 0, 0)),
            pl.BlockSpec((2, 8, 128), lambda i: (0, 0, 0)),
            pl.BlockSpec((2, 2, 128, 128), lambda i: (0, 0, 0, 0)),
        ],
        out_specs=[
            pl.BlockSpec((_B8, 128), lambda i: (i, 0)),
            pl.BlockSpec((2, _B8, 128), lambda i: (0, i, 0)),
        ],
        out_shape=[
            jax.ShapeDtypeStruct((rn, 128), jnp.float32),
            jax.ShapeDtypeStruct((2, rn, 128), jnp.float32),
        ],
    )(deg_pack, xt_pack, w1b, b1r, wcb)


def _tc_mid(acc_pack, g, dinv_pack, bprev_r, wcb_next):
    rn = g.shape[1]

    def body(acc_r, g_r, dinv_r, b_r, w_r, gout_r):
        dv = dinv_r[...]
        x0 = jnp.maximum(dv * (acc_r[0] + g_r[0]) + b_r[0, 0:1, :], 0.0)
        x1 = jnp.maximum(dv * (acc_r[1] + g_r[1]) + b_r[1, 0:1, :], 0.0)
        hw0 = jnp.dot(x0, w_r[0, 0], preferred_element_type=jnp.float32) + jnp.dot(
            x1, w_r[1, 0], preferred_element_type=jnp.float32
        )
        hw1 = jnp.dot(x0, w_r[0, 1], preferred_element_type=jnp.float32) + jnp.dot(
            x1, w_r[1, 1], preferred_element_type=jnp.float32
        )
        gout_r[0] = hw0 * dv
        gout_r[1] = hw1 * dv

    return pl.pallas_call(
        body,
        grid=(pl.cdiv(rn, _B8),),
        in_specs=[
            pl.BlockSpec((2, _B8, 128), lambda i: (0, i, 0)),
            pl.BlockSpec((2, _B8, 128), lambda i: (0, i, 0)),
            pl.BlockSpec((_B8, 128), lambda i: (i, 0)),
            pl.BlockSpec((2, 8, 128), lambda i: (0, 0, 0)),
            pl.BlockSpec((2, 2, 128, 128), lambda i: (0, 0, 0, 0)),
        ],
        out_specs=pl.BlockSpec((2, _B8, 128), lambda i: (0, i, 0)),
        out_shape=jax.ShapeDtypeStruct((2, rn, 128), jnp.float32),
    )(acc_pack, g, dinv_pack, bprev_r, wcb_next)


def _tc_fin(acc_pack, g, dinv_pack, bc3r, w3b, b3r):
    rn = g.shape[1]

    def body(acc_r, g_r, dinv_r, b_r, w_r, b3_r, y_r):
        dv = dinv_r[...]
        x0 = jnp.maximum(dv * (acc_r[0] + g_r[0]) + b_r[0, 0:1, :], 0.0)
        x1 = jnp.maximum(dv * (acc_r[1] + g_r[1]) + b_r[1, 0:1, :], 0.0)
        y = jnp.dot(x0, w_r[0], preferred_element_type=jnp.float32) + jnp.dot(
            x1, w_r[1], preferred_element_type=jnp.float32
        )
        y_r[...] = y + b3_r[0:1, :]

    return pl.pallas_call(
        body,
        grid=(pl.cdiv(rn, _B8),),
        in_specs=[
            pl.BlockSpec((2, _B8, 128), lambda i: (0, i, 0)),
            pl.BlockSpec((2, _B8, 128), lambda i: (0, i, 0)),
            pl.BlockSpec((_B8, 128), lambda i: (i, 0)),
            pl.BlockSpec((2, 8, 128), lambda i: (0, 0, 0)),
            pl.BlockSpec((2, 128, 8), lambda i: (0, 0, 0)),
            pl.BlockSpec((8, 8), lambda i: (0, 0)),
        ],
        out_specs=pl.BlockSpec((_B8, 8), lambda i: (i, 0)),
        out_shape=jax.ShapeDtypeStruct((rn, 8), jnp.float32),
    )(acc_pack, g, dinv_pack, bc3r, w3b, b3r)


def _kron8(w):
    return jnp.kron(jnp.eye(8, dtype=jnp.float32), w)


def _wc_blocks(wc):
    return jnp.stack(
        [
            jnp.stack([_kron8(wc[:16, :16]), _kron8(wc[:16, 16:])]),
            jnp.stack([_kron8(wc[16:, :16]), _kron8(wc[16:, 16:])]),
        ]
    )


def _b_rows(b):
    return jnp.stack(
        [
            jnp.tile(jnp.tile(b[:16], 8)[None, :], (8, 1)),
            jnp.tile(jnp.tile(b[16:], 8)[None, :], (8, 1)),
        ]
    )


def kernel(x, t, edge_index, W1, b1, Wc1, bc1, Wc2, bc2, Wc3, bc3, W3, b3):
    n = x.shape[0]
    e = edge_index.shape[1]

    src = edge_index[0].astype(jnp.int32)
    dst = edge_index[1].astype(jnp.int32)

    ch = -(-e // (NS * 128))
    ch = -(-ch // (2 * M)) * (2 * M)
    if NS * ch * 128 - e < M * 128:
        ch += 2 * M
    ep = NS * ch * 128
    pad = ep - e
    srcp = jnp.concatenate([src, jnp.zeros((pad,), jnp.int32)])
    dstp = jnp.concatenate([dst, jnp.full((pad,), n, jnp.int32)])
    srct = srcp.reshape(NS, ch, 128)
    dstt = dstp.reshape(NS, ch, 128)
    srcb = jnp.stack([srct, srct + n])

    np_ = ((n + 128) // 128) * 128
    zeros16 = jnp.zeros((np_, 16), jnp.float32)
    ones16 = jnp.ones((128, 16), jnp.float32)

    rn = n // 8
    xt_pack = jnp.stack([x, t], axis=1).reshape(rn, 16)
    w1b = jnp.stack([_kron8(W1[:, :16]), _kron8(W1[:, 16:])])
    b1r = _b_rows(b1)
    wcb1 = _wc_blocks(Wc1)
    wcb2 = _wc_blocks(Wc2)
    wcb3 = _wc_blocks(Wc3)
    bc1r = _b_rows(bc1)
    bc2r = _b_rows(bc2)
    bc3r = _b_rows(bc3)
    w3b = jnp.stack([_kron8(W3[:16, :]), _kron8(W3[16:, :])])
    b3r = jnp.tile(b3[None, :], (8, 8))

    deg_acc = _sc_deg(dstt, zeros16, ones16)
    dinv_pack, g = _tc0(deg_acc.reshape(NC, np_ // 8, 128), xt_pack, w1b, b1r, wcb1)
    acc1 = _sc_scatter(g.reshape(2 * n, 16), srcb, dstt, zeros16)
    g = _tc_mid(acc1.reshape(NC, np_ // 8, 128), g, dinv_pack, bc1r, wcb2)
    acc2 = _sc_scatter(g.reshape(2 * n, 16), srcb, dstt, zeros16)
    g = _tc_mid(acc2.reshape(NC, np_ // 8, 128), g, dinv_pack, bc2r, wcb3)
    acc3 = _sc_scatter(g.reshape(2 * n, 16), srcb, dstt, zeros16)
    y8 = _tc_fin(acc3.reshape(NC, np_ // 8, 128), g, dinv_pack, bc3r, w3b, b3r)
    return y8.reshape(-1)


# speedup vs baseline: 1.2593x; 1.0126x over previous
import functools

import jax
import jax.numpy as jnp
from jax import lax
from jax.experimental import pallas as pl
from jax.experimental.pallas import tpu as pltpu
from jax.experimental.pallas import tpu_sc as plsc

NC = 2
NS = 16
M = 4


def _sc_mesh():
    return plsc.VectorSubcoreMesh(
        core_axis_name="c", subcore_axis_name="s", num_cores=NC, num_subcores=NS
    )


def _sc_scatter(g2, srcb, dstt, zeros16):
    NP = zeros16.shape[0]
    CH = dstt.shape[1]
    NJ = CH // M
    RPT = NP // NS

    @functools.partial(
        pl.kernel,
        out_type=jax.ShapeDtypeStruct((NC, NP, 16), jnp.float32),
        mesh=_sc_mesh(),
        compiler_params=pltpu.CompilerParams(use_tc_tiling_on_sc=False),
        scratch_types=[
            pltpu.VMEM_SHARED((NP, 16), jnp.float32),
            pltpu.VMEM((3, M, 128), jnp.int32),
            pltpu.VMEM((3, M, 128), jnp.int32),
            pltpu.VMEM((3, M, 128, 16), jnp.float32),
            pltpu.SemaphoreType.DMA((3,)),
            pltpu.SemaphoreType.DMA,
            pltpu.SemaphoreType.DMA((3,)),
        ],
    )
    def k(g2_h, srcb_h, dstt_h, zeros_h, out_h, acc, sv, dv, rows, semg, sems, semi):
        c = lax.axis_index("c")
        s = lax.axis_index("s")
        r0 = s * RPT

        def fire_idx(j, slot):
            pltpu.async_copy(
                srcb_h.at[c, s, pl.ds(j * M, M)], sv.at[slot], semi.at[slot]
            )
            pltpu.async_copy(
                dstt_h.at[s, pl.ds(j * M, M)], dv.at[slot], semi.at[slot]
            )

        def wait_idx(slot):
            pltpu.make_async_copy(
                srcb_h.at[c, s, pl.ds(0, M)], sv.at[slot], semi.at[slot]
            ).wait()
            pltpu.make_async_copy(
                dstt_h.at[s, pl.ds(0, M)], dv.at[slot], semi.at[slot]
            ).wait()

        def fire_gathers(slot):
            for r in range(M):
                pltpu.async_copy(
                    g2_h.at[sv.at[slot, r]], rows.at[slot, r], semg.at[slot]
                )

        def drain_gathers(slot):
            for r in range(M):
                pltpu.make_async_copy(
                    g2_h.at[sv.at[slot, r]], rows.at[slot, r], semg.at[slot]
                ).wait()

        def fire_scatters(slot):
            for r in range(M):
                pltpu.async_copy(
                    rows.at[slot, r], acc.at[dv.at[slot, r]], sems, add=True
                )

        def drain_scatters(slot):
            for r in range(M):
                pltpu.make_async_copy(
                    rows.at[slot, r], acc.at[dv.at[slot, r]], sems
                ).wait()

        pltpu.sync_copy(dstt_h.at[NS - 1, pl.ds(CH - M, M)], dv.at[2])
        fire_scatters(2)
        fire_idx(0, 0)
        fire_idx(1, 1)
        pltpu.sync_copy(zeros_h.at[pl.ds(r0, RPT)], acc.at[pl.ds(r0, RPT)])
        plsc.subcore_barrier()
        wait_idx(0)
        fire_gathers(0)

        def body(j, carry):
            a = lax.rem(j, 3)
            b = lax.rem(j + 1, 3)
            f = lax.rem(j + 2, 3)
            drain_scatters(f)
            fire_idx(lax.rem(j + 2, NJ), f)
            wait_idx(b)
            fire_gathers(b)
            for r in range(M):
                pltpu.make_async_copy(
                    g2_h.at[sv.at[a, r]], rows.at[a, r], semg.at[a]
                ).wait()
                pltpu.async_copy(
                    rows.at[a, r], acc.at[dv.at[a, r]], sems, add=True
                )
            return carry

        lax.fori_loop(0, NJ, body, 0)
        drain_scatters((NJ - 1) % 3)
        drain_gathers(NJ % 3)
        wait_idx((NJ + 1) % 3)
        plsc.subcore_barrier()
        pltpu.sync_copy(acc.at[pl.ds(r0, RPT)], out_h.at[c, pl.ds(r0, RPT)])

    return k(g2, srcb, dstt, zeros16)


def _sc_deg(dstt, zeros16, ones16):
    NP = zeros16.shape[0]
    CH = dstt.shape[1]
    HALF = CH // 2
    MD = 8
    NJ = HALF // MD
    RPT = NP // NS

    @functools.partial(
        pl.kernel,
        out_type=jax.ShapeDtypeStruct((NC, NP, 16), jnp.float32),
        mesh=_sc_mesh(),
        compiler_params=pltpu.CompilerParams(use_tc_tiling_on_sc=False),
        scratch_types=[
            pltpu.VMEM_SHARED((NP, 16), jnp.float32),
            pltpu.VMEM((2, MD, 128), jnp.int32),
            pltpu.VMEM((128, 16), jnp.float32),
            pltpu.SemaphoreType.DMA,
            pltpu.SemaphoreType.DMA((2,)),
        ],
    )
    def k(dstt_h, zeros_h, ones_h, out_h, acc, dv, ones_v, sems, semi):
        c = lax.axis_index("c")
        s = lax.axis_index("s")
        r0 = s * RPT
        pltpu.sync_copy(ones_h, ones_v)
        pltpu.sync_copy(dstt_h.at[NS - 1, pl.ds(CH - MD, MD)], dv.at[1])
        for r in range(MD):
            pltpu.async_copy(ones_v, acc.at[dv.at[1, r]], sems, add=True)
        pltpu.async_copy(dstt_h.at[s, pl.ds(c * HALF, MD)], dv.at[0], semi.at[0])
        pltpu.sync_copy(zeros_h.at[pl.ds(r0, RPT)], acc.at[pl.ds(r0, RPT)])
        plsc.subcore_barrier()

        def body(j, carry):
            p = lax.rem(j, 2)
            q = 1 - p
            for r in range(MD):
                pltpu.make_async_copy(ones_v, acc.at[dv.at[q, r]], sems).wait()
            jn = lax.rem(j + 1, NJ)
            pltpu.async_copy(
                dstt_h.at[s, pl.ds(c * HALF + jn * MD, MD)], dv.at[q], semi.at[q]
            )
            pltpu.make_async_copy(
                dstt_h.at[s, pl.ds(c * HALF + j * MD, MD)], dv.at[p], semi.at[p]
            ).wait()
            for r in range(MD):
                pltpu.async_copy(ones_v, acc.at[dv.at[p, r]], sems, add=True)
            return carry

        lax.fori_loop(0, NJ, body, 0)
        pf = NJ % 2
        pltpu.make_async_copy(
            dstt_h.at[s, pl.ds(c * HALF, MD)], dv.at[pf], semi.at[pf]
        ).wait()
        for r in range(MD):
            pltpu.make_async_copy(
                ones_v, acc.at[dv.at[(NJ - 1) % 2, r]], sems
            ).wait()
        plsc.subcore_barrier()
        pltpu.sync_copy(acc.at[pl.ds(r0, RPT)], out_h.at[c, pl.ds(r0, RPT)])

    return k(dstt, zeros16, ones16)


_B8 = 2048


def _tc0(deg_pack, xt_pack, w1b, b1r, wcb):
    rn = xt_pack.shape[0]

    def body(deg_r, xt_r, w1_r, b1_r, wc_r, dinv_r, g_r):
        deg = deg_r[0] + deg_r[1] + 1.0
        dinv = lax.rsqrt(deg)
        x0 = jnp.maximum(
            jnp.dot(xt_r[...], w1_r[0], preferred_element_type=jnp.float32)
            + b1_r[0, 0:1, :],
            0.0,
        )
        x1 = jnp.maximum(
            jnp.dot(xt_r[...], w1_r[1], preferred_element_type=jnp.float32)
            + b1_r[1, 0:1, :],
            0.0,
        )
        hw0 = jnp.dot(x0, wc_r[0, 0], preferred_element_type=jnp.float32) + jnp.dot(
            x1, wc_r[1, 0], preferred_element_type=jnp.float32
        )
        hw1 = jnp.dot(x0, wc_r[0, 1], preferred_element_type=jnp.float32) + jnp.dot(
            x1, wc_r[1, 1], preferred_element_type=jnp.float32
        )
        dinv_r[...] = dinv
        g_r[0] = hw0 * dinv
        g_r[1] = hw1 * dinv

    return pl.pallas_call(
        body,
        grid=(pl.cdiv(rn, _B8),),
        in_specs=[
            pl.BlockSpec((2, _B8, 128), lambda i: (0, i, 0)),
            pl.BlockSpec((_B8, 16), lambda i: (i, 0)),
            pl.BlockSpec((2, 16, 128), lambda i: (0, 0, 0)),
            pl.BlockSpec((2, 8, 128), lambda i: (0, 0, 0)),
            pl.BlockSpec((2, 2, 128, 128), lambda i: (0, 0, 0, 0)),
        ],
        out_specs=[
            pl.BlockSpec((_B8, 128), lambda i: (i, 0)),
            pl.BlockSpec((2, _B8, 128), lambda i: (0, i, 0)),
        ],
        out_shape=[
            jax.ShapeDtypeStruct((rn, 128), jnp.float32),
            jax.ShapeDtypeStruct((2, rn, 128), jnp.float32),
        ],
    )(deg_pack, xt_pack, w1b, b1r, wcb)


def _tc_mid(acc_pack, g, dinv_pack, bprev_r, wcb_next):
    rn = g.shape[1]

    def body(acc_r, g_r, dinv_r, b_r, w_r, gout_r):
        dv = dinv_r[...]
        x0 = jnp.maximum(dv * (acc_r[0] + g_r[0]) + b_r[0, 0:1, :], 0.0)
        x1 = jnp.maximum(dv * (acc_r[1] + g_r[1]) + b_r[1, 0:1, :], 0.0)
        hw0 = jnp.dot(x0, w_r[0, 0], preferred_element_type=jnp.float32) + jnp.dot(
            x1, w_r[1, 0], preferred_element_type=jnp.float32
        )
        hw1 = jnp.dot(x0, w_r[0, 1], preferred_element_type=jnp.float32) + jnp.dot(
            x1, w_r[1, 1], preferred_element_type=jnp.float32
        )
        gout_r[0] = hw0 * dv
        gout_r[1] = hw1 * dv

    return pl.pallas_call(
        body,
        grid=(pl.cdiv(rn, _B8),),
        in_specs=[
            pl.BlockSpec((2, _B8, 128), lambda i: (0, i, 0)),
            pl.BlockSpec((2, _B8, 128), lambda i: (0, i, 0)),
            pl.BlockSpec((_B8, 128), lambda i: (i, 0)),
            pl.BlockSpec((2, 8, 128), lambda i: (0, 0, 0)),
            pl.BlockSpec((2, 2, 128, 128), lambda i: (0, 0, 0, 0)),
        ],
        out_specs=pl.BlockSpec((2, _B8, 128), lambda i: (0, i, 0)),
        out_shape=jax.ShapeDtypeStruct((2, rn, 128), jnp.float32),
    )(acc_pack, g, dinv_pack, bprev_r, wcb_next)


def _tc_fin(acc_pack, g, dinv_pack, bc3r, w3b, b3r):
    rn = g.shape[1]

    def body(acc_r, g_r, dinv_r, b_r, w_r, b3_r, y_r):
        dv = dinv_r[...]
        x0 = jnp.maximum(dv * (acc_r[0] + g_r[0]) + b_r[0, 0:1, :], 0.0)
        x1 = jnp.maximum(dv * (acc_r[1] + g_r[1]) + b_r[1, 0:1, :], 0.0)
        y = jnp.dot(x0, w_r[0], preferred_element_type=jnp.float32) + jnp.dot(
            x1, w_r[1], preferred_element_type=jnp.float32
        )
        y_r[...] = y + b3_r[0:1, :]

    return pl.pallas_call(
        body,
        grid=(pl.cdiv(rn, _B8),),
        in_specs=[
            pl.BlockSpec((2, _B8, 128), lambda i: (0, i, 0)),
            pl.BlockSpec((2, _B8, 128), lambda i: (0, i, 0)),
            pl.BlockSpec((_B8, 128), lambda i: (i, 0)),
            pl.BlockSpec((2, 8, 128), lambda i: (0, 0, 0)),
            pl.BlockSpec((2, 128, 8), lambda i: (0, 0, 0)),
            pl.BlockSpec((8, 8), lambda i: (0, 0)),
        ],
        out_specs=pl.BlockSpec((_B8, 8), lambda i: (i, 0)),
        out_shape=jax.ShapeDtypeStruct((rn, 8), jnp.float32),
    )(acc_pack, g, dinv_pack, bc3r, w3b, b3r)


def _kron8(w):
    return jnp.kron(jnp.eye(8, dtype=jnp.float32), w)


def _wc_blocks(wc):
    return jnp.stack(
        [
            jnp.stack([_kron8(wc[:16, :16]), _kron8(wc[:16, 16:])]),
            jnp.stack([_kron8(wc[16:, :16]), _kron8(wc[16:, 16:])]),
        ]
    )


def _b_rows(b):
    return jnp.stack(
        [
            jnp.tile(jnp.tile(b[:16], 8)[None, :], (8, 1)),
            jnp.tile(jnp.tile(b[16:], 8)[None, :], (8, 1)),
        ]
    )


def kernel(x, t, edge_index, W1, b1, Wc1, bc1, Wc2, bc2, Wc3, bc3, W3, b3):
    n = x.shape[0]
    e = edge_index.shape[1]

    src = edge_index[0].astype(jnp.int32)
    dst = edge_index[1].astype(jnp.int32)

    ch = -(-e // (NS * 128))
    ch = -(-ch // (2 * M)) * (2 * M)
    if NS * ch * 128 - e < M * 128:
        ch += 2 * M
    ep = NS * ch * 128
    pad = ep - e
    srcp = jnp.concatenate([src, jnp.zeros((pad,), jnp.int32)])
    dstp = jnp.concatenate([dst, jnp.full((pad,), n, jnp.int32)])
    srct = srcp.reshape(NS, ch, 128)
    dstt = dstp.reshape(NS, ch, 128)
    srcb = jnp.stack([srct, srct + n])

    np_ = ((n + 128) // 128) * 128
    zeros16 = jnp.zeros((np_, 16), jnp.float32)
    ones16 = jnp.ones((128, 16), jnp.float32)

    rn = n // 8
    xt_pack = jnp.stack([x, t], axis=1).reshape(rn, 16)
    w1b = jnp.stack([_kron8(W1[:, :16]), _kron8(W1[:, 16:])])
    b1r = _b_rows(b1)
    wcb1 = _wc_blocks(Wc1)
    wcb2 = _wc_blocks(Wc2)
    wcb3 = _wc_blocks(Wc3)
    bc1r = _b_rows(bc1)
    bc2r = _b_rows(bc2)
    bc3r = _b_rows(bc3)
    w3b = jnp.stack([_kron8(W3[:16, :]), _kron8(W3[16:, :])])
    b3r = jnp.tile(b3[None, :], (8, 8))

    deg_acc = _sc_deg(dstt, zeros16, ones16)
    dinv_pack, g = _tc0(deg_acc.reshape(NC, np_ // 8, 128), xt_pack, w1b, b1r, wcb1)
    acc1 = _sc_scatter(g.reshape(2 * n, 16), srcb, dstt, zeros16)
    g = _tc_mid(acc1.reshape(NC, np_ // 8, 128), g, dinv_pack, bc1r, wcb2)
    acc2 = _sc_scatter(g.reshape(2 * n, 16), srcb, dstt, zeros16)
    g = _tc_mid(acc2.reshape(NC, np_ // 8, 128), g, dinv_pack, bc2r, wcb3)
    acc3 = _sc_scatter(g.reshape(2 * n, 16), srcb, dstt, zeros16)
    y8 = _tc_fin(acc3.reshape(NC, np_ // 8, 128), g, dinv_pack, bc3r, w3b, b3r)
    return y8.reshape(-1)
